# baseline (speedup 1.0000x reference)
import functools

import jax
import jax.numpy as jnp
from jax import lax
from jax.experimental import pallas as pl
from jax.experimental.pallas import tpu as pltpu
from jax.experimental.pallas import tpu_sc as plsc

N = 10000
E = 320000
D = 128

NC = 2
NS = 16
NW = NC * NS
CH = 96
NCHUNK = 105
EPW = NCHUNK * CH
EPAD = NW * EPW
NPAD = 10240
RPS = NPAD // NS

_sc_mesh = plsc.VectorSubcoreMesh(core_axis_name="c", subcore_axis_name="s")


@functools.partial(
    pl.kernel,
    out_type=jax.ShapeDtypeStruct((NC, NPAD, D), jnp.float32),
    mesh=_sc_mesh,
    scratch_types=[
        pltpu.VMEM((EPW,), jnp.int32),
        pltpu.VMEM((NCHUNK, CH), jnp.int32),
        pltpu.VMEM((CH, D), jnp.float32),
        pltpu.VMEM((CH, D), jnp.float32),
        pltpu.SemaphoreType.DMA,
        pltpu.SemaphoreType.DMA,
        pltpu.SemaphoreType.DMA,
        pltpu.SemaphoreType.DMA,
        pltpu.VMEM_SHARED((NPAD, D), jnp.float32),
    ],
)
def _sc_agg(x_hbm, src_hbm, dst_hbm, zeros_hbm, out_hbm,
            src_v, dst_v, rows0, rows1, sem0, sem1, ssem0, ssem1, agg_sh):
    cid = lax.axis_index("c")
    sid = lax.axis_index("s")
    wid = cid * NS + sid
    eb = wid * EPW

    r0 = sid * RPS
    pltpu.sync_copy(zeros_hbm.at[pl.ds(r0, RPS)], agg_sh.at[pl.ds(r0, RPS)])
    pltpu.sync_copy(src_hbm.at[pl.ds(eb, EPW)], src_v)
    pltpu.sync_copy(dst_hbm.at[wid], dst_v)
    plsc.subcore_barrier()

    pltpu.async_copy(x_hbm.at[src_v.at[pl.ds(0, CH)]], rows0, sem0)
    pltpu.async_copy(x_hbm.at[src_v.at[pl.ds(CH, CH)]], rows1, sem1)

    @pl.loop(0, NCHUNK)
    def _(i):
        base = i * CH

        @pl.when(lax.rem(i, 2) == 0)
        def _():
            pltpu.make_async_copy(
                x_hbm.at[src_v.at[pl.ds(base, CH)]], rows0, sem0).wait()
            pltpu.async_copy(rows0, agg_sh.at[dst_v.at[i]], ssem0, add=True)

            @pl.when(i >= 1)
            def _():
                pltpu.make_async_copy(
                    rows1, agg_sh.at[dst_v.at[i - 1]], ssem1).wait()

                @pl.when(i + 1 < NCHUNK)
                def _():
                    pltpu.async_copy(
                        x_hbm.at[src_v.at[pl.ds(base + CH, CH)]], rows1, sem1)

        @pl.when(lax.rem(i, 2) == 1)
        def _():
            pltpu.make_async_copy(
                x_hbm.at[src_v.at[pl.ds(base, CH)]], rows1, sem1).wait()
            pltpu.async_copy(rows1, agg_sh.at[dst_v.at[i]], ssem1, add=True)

            pltpu.make_async_copy(
                rows0, agg_sh.at[dst_v.at[i - 1]], ssem0).wait()

            @pl.when(i + 1 < NCHUNK)
            def _():
                pltpu.async_copy(
                    x_hbm.at[src_v.at[pl.ds(base + CH, CH)]], rows0, sem0)

    pltpu.make_async_copy(rows0, agg_sh.at[dst_v.at[NCHUNK - 1]], ssem0).wait()
    plsc.subcore_barrier()
    pltpu.sync_copy(agg_sh.at[pl.ds(r0, RPS)], out_hbm.at[cid, pl.ds(r0, RPS)])


_BLK = 1000


def _mlp1_body(x_ref, agg_ref, w_ref, b_ref, o_ref):
    h = x_ref[...] + agg_ref[0] + agg_ref[1]
    z = lax.dot_general(h, w_ref[...], (((1,), (1,)), ((), ())),
                        precision=lax.Precision.HIGHEST,
                        preferred_element_type=jnp.float32)
    o_ref[...] = jnp.maximum(z + b_ref[...], 0.0)


def _mlp2_body(h_ref, agg_ref, w_ref, b_ref, o_ref):
    h = h_ref[...] + agg_ref[0] + agg_ref[1]
    z = lax.dot_general(h, w_ref[...], (((1,), (1,)), ((), ())),
                        precision=lax.Precision.HIGHEST,
                        preferred_element_type=jnp.float32)
    z = z + b_ref[...]
    m = jnp.max(z, axis=1, keepdims=True)
    lse = m + jnp.log(jnp.sum(jnp.exp(z - m), axis=1, keepdims=True))
    o_ref[...] = z - lse


def _make_mlp(body):
    return pl.pallas_call(
        body,
        grid=(N // _BLK,),
        in_specs=[
            pl.BlockSpec((_BLK, D), lambda i: (i, 0)),
            pl.BlockSpec((NC, _BLK, D), lambda i: (0, i, 0)),
            pl.BlockSpec((D, D), lambda i: (0, 0)),
            pl.BlockSpec((1, D), lambda i: (0, 0)),
        ],
        out_specs=pl.BlockSpec((_BLK, D), lambda i: (i, 0)),
        out_shape=jax.ShapeDtypeStruct((N, D), jnp.float32),
    )


_mlp1 = _make_mlp(_mlp1_body)
_mlp2 = _make_mlp(_mlp2_body)


def kernel(x, edge_index, W1, b1, W2, b2):
    npad_e = EPAD - E
    src = jnp.concatenate([edge_index[0], jnp.zeros((npad_e,), jnp.int32)])
    dst = jnp.concatenate(
        [edge_index[1], jnp.full((npad_e,), N, jnp.int32)]
    ).reshape(NW, NCHUNK, CH)
    zeros = jnp.zeros((NPAD, D), jnp.float32)
    b1r = b1.reshape(1, D)
    b2r = b2.reshape(1, D)

    agg1 = _sc_agg(x, src, dst, zeros)
    h1 = _mlp1(x, agg1, W1, b1r)
    agg2 = _sc_agg(h1, src, dst, zeros)
    return _mlp2(h1, agg2, W2, b2r)


# speedup vs baseline: 1.1074x; 1.1074x over previous
import functools

import jax
import jax.numpy as jnp
from jax import lax
from jax.experimental import pallas as pl
from jax.experimental.pallas import tpu as pltpu
from jax.experimental.pallas import tpu_sc as plsc

N = 10000
E = 320000
D = 128

NC = 2
NS = 16
NW = NC * NS
CH = 96
NCHUNK = 105
EPW = NCHUNK * CH
EPAD = NW * EPW
NPAD = 10240
RPS = NPAD // NS

_sc_mesh = plsc.VectorSubcoreMesh(core_axis_name="c", subcore_axis_name="s")


@functools.partial(
    pl.kernel,
    out_type=jax.ShapeDtypeStruct((NC, NPAD, D), jnp.float32),
    mesh=_sc_mesh,
    scratch_types=[
        pltpu.VMEM((EPW,), jnp.int32),
        pltpu.VMEM((NCHUNK, CH), jnp.int32),
        pltpu.VMEM((CH, D), jnp.float32),
        pltpu.VMEM((CH, D), jnp.float32),
        pltpu.SemaphoreType.DMA,
        pltpu.SemaphoreType.DMA,
        pltpu.VMEM_SHARED((NPAD, D), jnp.float32),
    ],
)
def _sc_agg(x_hbm, src_hbm, dst_hbm, zeros_hbm, out_hbm,
            src_v, dst_v, rows0, rows1, sem0, sem1, agg_sh):
    cid = lax.axis_index("c")
    sid = lax.axis_index("s")
    wid = cid * NS + sid
    eb = wid * EPW

    r0 = sid * RPS
    pltpu.sync_copy(zeros_hbm.at[pl.ds(r0, RPS)], agg_sh.at[pl.ds(r0, RPS)])
    pltpu.sync_copy(src_hbm.at[pl.ds(eb, EPW)], src_v)
    pltpu.sync_copy(dst_hbm.at[wid], dst_v)
    plsc.subcore_barrier()

    pltpu.async_copy(x_hbm.at[src_v.at[pl.ds(0, CH)]], rows0, sem0)
    pltpu.async_copy(x_hbm.at[src_v.at[pl.ds(CH, CH)]], rows1, sem1)

    @pl.loop(0, NCHUNK)
    def _(i):
        base = i * CH

        @pl.when(lax.rem(i, 2) == 0)
        def _():
            pltpu.make_async_copy(
                x_hbm.at[src_v.at[pl.ds(base, CH)]], rows0, sem0).wait()
            pltpu.sync_copy(rows0, agg_sh.at[dst_v.at[i]], add=True)

            @pl.when(i + 2 < NCHUNK)
            def _():
                pltpu.async_copy(
                    x_hbm.at[src_v.at[pl.ds(base + 2 * CH, CH)]], rows0, sem0)

        @pl.when(lax.rem(i, 2) == 1)
        def _():
            pltpu.make_async_copy(
                x_hbm.at[src_v.at[pl.ds(base, CH)]], rows1, sem1).wait()
            pltpu.sync_copy(rows1, agg_sh.at[dst_v.at[i]], add=True)

            @pl.when(i + 2 < NCHUNK)
            def _():
                pltpu.async_copy(
                    x_hbm.at[src_v.at[pl.ds(base + 2 * CH, CH)]], rows1, sem1)

    plsc.subcore_barrier()
    pltpu.sync_copy(agg_sh.at[pl.ds(r0, RPS)], out_hbm.at[cid, pl.ds(r0, RPS)])


_BLK = 1000


def _mlp1_body(x_ref, agg_ref, w_ref, b_ref, o_ref):
    h = x_ref[...] + agg_ref[0] + agg_ref[1]
    z = lax.dot_general(h, w_ref[...], (((1,), (1,)), ((), ())),
                        precision=lax.Precision.HIGHEST,
                        preferred_element_type=jnp.float32)
    o_ref[...] = jnp.maximum(z + b_ref[...], 0.0)


def _mlp2_body(h_ref, agg_ref, w_ref, b_ref, o_ref):
    h = h_ref[...] + agg_ref[0] + agg_ref[1]
    z = lax.dot_general(h, w_ref[...], (((1,), (1,)), ((), ())),
                        precision=lax.Precision.HIGHEST,
                        preferred_element_type=jnp.float32)
    z = z + b_ref[...]
    m = jnp.max(z, axis=1, keepdims=True)
    lse = m + jnp.log(jnp.sum(jnp.exp(z - m), axis=1, keepdims=True))
    o_ref[...] = z - lse


def _make_mlp(body):
    return pl.pallas_call(
        body,
        grid=(N // _BLK,),
        in_specs=[
            pl.BlockSpec((_BLK, D), lambda i: (i, 0)),
            pl.BlockSpec((NC, _BLK, D), lambda i: (0, i, 0)),
            pl.BlockSpec((D, D), lambda i: (0, 0)),
            pl.BlockSpec((1, D), lambda i: (0, 0)),
        ],
        out_specs=pl.BlockSpec((_BLK, D), lambda i: (i, 0)),
        out_shape=jax.ShapeDtypeStruct((N, D), jnp.float32),
    )


_mlp1 = _make_mlp(_mlp1_body)
_mlp2 = _make_mlp(_mlp2_body)


def kernel(x, edge_index, W1, b1, W2, b2):
    npad_e = EPAD - E
    src = jnp.concatenate([edge_index[0], jnp.zeros((npad_e,), jnp.int32)])
    dst = jnp.concatenate(
        [edge_index[1], jnp.full((npad_e,), N, jnp.int32)]
    ).reshape(NW, NCHUNK, CH)
    zeros = jnp.zeros((NPAD, D), jnp.float32)
    b1r = b1.reshape(1, D)
    b2r = b2.reshape(1, D)

    agg1 = _sc_agg(x, src, dst, zeros)
    h1 = _mlp1(x, agg1, W1, b1r)
    agg2 = _sc_agg(h1, src, dst, zeros)
    return _mlp2(h1, agg2, W2, b2r)


# speedup vs baseline: 1.9204x; 1.7342x over previous
import functools

import jax
import jax.numpy as jnp
from jax import lax
from jax.experimental import pallas as pl
from jax.experimental.pallas import tpu as pltpu
from jax.experimental.pallas import tpu_sc as plsc

N = 10000
E = 320000
D = 128

NC = 2
NS = 16
NW = NC * NS
CH = 96
NCHUNK = 105
EPW = NCHUNK * CH
EPAD = NW * EPW
NPAD = 10240
RPS = NPAD // NS

_sc_mesh = plsc.VectorSubcoreMesh(core_axis_name="c", subcore_axis_name="s")


@functools.partial(
    pl.kernel,
    out_type=jax.ShapeDtypeStruct((NC, NPAD, D), jnp.float32),
    mesh=_sc_mesh,
    scratch_types=[
        pltpu.VMEM((EPW,), jnp.int32),
        pltpu.VMEM((NCHUNK, CH), jnp.int32),
        pltpu.VMEM((CH, D), jnp.float32),
        pltpu.VMEM((CH, D), jnp.float32),
        pltpu.SemaphoreType.DMA,
        pltpu.SemaphoreType.DMA,
        pltpu.VMEM_SHARED((NPAD, D), jnp.float32),
    ],
)
def _sc_agg(x_hbm, src_hbm, dst_hbm, zeros_hbm, out_hbm,
            src_v, dst_v, rows0, rows1, sem0, sem1, agg_sh):
    cid = lax.axis_index("c")
    sid = lax.axis_index("s")
    wid = cid * NS + sid
    eb = wid * EPW

    r0 = sid * RPS
    pltpu.sync_copy(zeros_hbm.at[pl.ds(r0, RPS)], agg_sh.at[pl.ds(r0, RPS)])
    pltpu.sync_copy(src_hbm.at[pl.ds(eb, EPW)], src_v)
    pltpu.sync_copy(dst_hbm.at[wid], dst_v)
    plsc.subcore_barrier()

    pltpu.async_copy(x_hbm.at[src_v.at[pl.ds(0, CH)]], rows0, sem0)
    pltpu.async_copy(x_hbm.at[src_v.at[pl.ds(CH, CH)]], rows1, sem1)

    @pl.loop(0, NCHUNK)
    def _(i):
        base = i * CH

        @pl.when(lax.rem(i, 2) == 0)
        def _():
            pltpu.make_async_copy(
                x_hbm.at[src_v.at[pl.ds(base, CH)]], rows0, sem0).wait()
            pltpu.sync_copy(rows0, agg_sh.at[dst_v.at[i]], add=True)

            @pl.when(i + 2 < NCHUNK)
            def _():
                pltpu.async_copy(
                    x_hbm.at[src_v.at[pl.ds(base + 2 * CH, CH)]], rows0, sem0)

        @pl.when(lax.rem(i, 2) == 1)
        def _():
            pltpu.make_async_copy(
                x_hbm.at[src_v.at[pl.ds(base, CH)]], rows1, sem1).wait()
            pltpu.sync_copy(rows1, agg_sh.at[dst_v.at[i]], add=True)

            @pl.when(i + 2 < NCHUNK)
            def _():
                pltpu.async_copy(
                    x_hbm.at[src_v.at[pl.ds(base + 2 * CH, CH)]], rows1, sem1)

    plsc.subcore_barrier()
    pltpu.sync_copy(agg_sh.at[pl.ds(r0, RPS)], out_hbm.at[cid, pl.ds(r0, RPS)])


_BLK = 1000


def _mlp1_body(x_ref, agg_ref, w_ref, b_ref, o_ref):
    h = x_ref[...] + agg_ref[0] + agg_ref[1]
    z = lax.dot_general(h, w_ref[...], (((1,), (1,)), ((), ())),
                        precision=lax.Precision.HIGHEST,
                        preferred_element_type=jnp.float32)
    o_ref[...] = jnp.maximum(z + b_ref[...], 0.0)


def _mlp2_body(h_ref, agg_ref, w_ref, b_ref, o_ref):
    h = h_ref[...] + agg_ref[0] + agg_ref[1]
    z = lax.dot_general(h, w_ref[...], (((1,), (1,)), ((), ())),
                        precision=lax.Precision.HIGHEST,
                        preferred_element_type=jnp.float32)
    z = z + b_ref[...]
    m = jnp.max(z, axis=1, keepdims=True)
    lse = m + jnp.log(jnp.sum(jnp.exp(z - m), axis=1, keepdims=True))
    o_ref[...] = z - lse


def _make_mlp(body):
    return pl.pallas_call(
        body,
        grid=(N // _BLK,),
        in_specs=[
            pl.BlockSpec((_BLK, D), lambda i: (i, 0)),
            pl.BlockSpec((NC, _BLK, D), lambda i: (0, i, 0)),
            pl.BlockSpec((D, D), lambda i: (0, 0)),
            pl.BlockSpec((1, D), lambda i: (0, 0)),
        ],
        out_specs=pl.BlockSpec((_BLK, D), lambda i: (i, 0)),
        out_shape=jax.ShapeDtypeStruct((N, D), jnp.float32),
    )


_mlp1 = _make_mlp(_mlp1_body)
_mlp2 = _make_mlp(_mlp2_body)


def kernel(x, edge_index, W1, b1, W2, b2):
    npad_e = EPAD - E
    pad_ids = jnp.arange(npad_e, dtype=jnp.int32)
    src = jnp.concatenate([edge_index[0], pad_ids % N])
    dst = jnp.concatenate(
        [edge_index[1], N + pad_ids % (NPAD - N)]
    ).reshape(NW, NCHUNK, CH)
    zeros = jnp.zeros((NPAD, D), jnp.float32)
    b1r = b1.reshape(1, D)
    b2r = b2.reshape(1, D)

    agg1 = _sc_agg(x, src, dst, zeros)
    h1 = _mlp1(x, agg1, W1, b1r)
    agg2 = _sc_agg(h1, src, dst, zeros)
    return _mlp2(h1, agg2, W2, b2r)


# speedup vs baseline: 1.9475x; 1.0141x over previous
import functools

import jax
import jax.numpy as jnp
from jax import lax
from jax.experimental import pallas as pl
from jax.experimental.pallas import tpu as pltpu
from jax.experimental.pallas import tpu_sc as plsc

N = 10000
E = 320000
D = 128

NC = 2
NS = 16
NW = NC * NS
CH = 96
NCHUNK = 105
EPW = NCHUNK * CH
EPAD = NW * EPW
NPAD = 10240
RPS = NPAD // NS

_sc_mesh = plsc.VectorSubcoreMesh(core_axis_name="c", subcore_axis_name="s")


@functools.partial(
    pl.kernel,
    out_type=jax.ShapeDtypeStruct((NC, NPAD, D), jnp.float32),
    mesh=_sc_mesh,
    scratch_types=[
        pltpu.VMEM((EPW,), jnp.int32),
        pltpu.VMEM((NCHUNK, CH), jnp.int32),
        pltpu.VMEM((CH, D), jnp.float32),
        pltpu.VMEM((CH, D), jnp.float32),
        pltpu.SemaphoreType.DMA,
        pltpu.SemaphoreType.DMA,
        pltpu.VMEM_SHARED((NPAD, D), jnp.float32),
    ],
)
def _sc_agg(x_hbm, src_hbm, dst_hbm, zeros_hbm, out_hbm,
            src_v, dst_v, rows0, rows1, sem0, sem1, agg_sh):
    cid = lax.axis_index("c")
    sid = lax.axis_index("s")
    wid = cid * NS + sid
    eb = wid * EPW

    r0 = sid * RPS
    pltpu.async_copy(
        zeros_hbm.at[pl.ds(r0, RPS)], agg_sh.at[pl.ds(r0, RPS)], sem0)
    pltpu.async_copy(src_hbm.at[pl.ds(eb, EPW)], src_v, sem1)
    pltpu.sync_copy(dst_hbm.at[wid], dst_v)
    pltpu.make_async_copy(
        zeros_hbm.at[pl.ds(r0, RPS)], agg_sh.at[pl.ds(r0, RPS)], sem0).wait()
    pltpu.make_async_copy(src_hbm.at[pl.ds(eb, EPW)], src_v, sem1).wait()
    plsc.subcore_barrier()

    pltpu.async_copy(x_hbm.at[src_v.at[pl.ds(0, CH)]], rows0, sem0)
    pltpu.async_copy(x_hbm.at[src_v.at[pl.ds(CH, CH)]], rows1, sem1)

    @pl.loop(0, NCHUNK)
    def _(i):
        base = i * CH

        @pl.when(lax.rem(i, 2) == 0)
        def _():
            pltpu.make_async_copy(
                x_hbm.at[src_v.at[pl.ds(base, CH)]], rows0, sem0).wait()
            pltpu.sync_copy(rows0, agg_sh.at[dst_v.at[i]], add=True)

            @pl.when(i + 2 < NCHUNK)
            def _():
                pltpu.async_copy(
                    x_hbm.at[src_v.at[pl.ds(base + 2 * CH, CH)]], rows0, sem0)

        @pl.when(lax.rem(i, 2) == 1)
        def _():
            pltpu.make_async_copy(
                x_hbm.at[src_v.at[pl.ds(base, CH)]], rows1, sem1).wait()
            pltpu.sync_copy(rows1, agg_sh.at[dst_v.at[i]], add=True)

            @pl.when(i + 2 < NCHUNK)
            def _():
                pltpu.async_copy(
                    x_hbm.at[src_v.at[pl.ds(base + 2 * CH, CH)]], rows1, sem1)

    plsc.subcore_barrier()
    pltpu.sync_copy(agg_sh.at[pl.ds(r0, RPS)], out_hbm.at[cid, pl.ds(r0, RPS)])


_BLK = 1000


def _mlp1_body(x_ref, agg_ref, w_ref, b_ref, o_ref):
    h = x_ref[...] + agg_ref[0] + agg_ref[1]
    z = lax.dot_general(h, w_ref[...], (((1,), (1,)), ((), ())),
                        precision=lax.Precision.HIGHEST,
                        preferred_element_type=jnp.float32)
    o_ref[...] = jnp.maximum(z + b_ref[...], 0.0)


def _mlp2_body(h_ref, agg_ref, w_ref, b_ref, o_ref):
    h = h_ref[...] + agg_ref[0] + agg_ref[1]
    z = lax.dot_general(h, w_ref[...], (((1,), (1,)), ((), ())),
                        precision=lax.Precision.HIGHEST,
                        preferred_element_type=jnp.float32)
    z = z + b_ref[...]
    m = jnp.max(z, axis=1, keepdims=True)
    lse = m + jnp.log(jnp.sum(jnp.exp(z - m), axis=1, keepdims=True))
    o_ref[...] = z - lse


def _make_mlp(body):
    return pl.pallas_call(
        body,
        grid=(N // _BLK,),
        in_specs=[
            pl.BlockSpec((_BLK, D), lambda i: (i, 0)),
            pl.BlockSpec((NC, _BLK, D), lambda i: (0, i, 0)),
            pl.BlockSpec((D, D), lambda i: (0, 0)),
            pl.BlockSpec((1, D), lambda i: (0, 0)),
        ],
        out_specs=pl.BlockSpec((_BLK, D), lambda i: (i, 0)),
        out_shape=jax.ShapeDtypeStruct((N, D), jnp.float32),
    )


_mlp1 = _make_mlp(_mlp1_body)
_mlp2 = _make_mlp(_mlp2_body)


def kernel(x, edge_index, W1, b1, W2, b2):
    npad_e = EPAD - E
    pad_ids = jnp.arange(npad_e, dtype=jnp.int32)
    src = jnp.concatenate([edge_index[0], pad_ids % N])
    dst = jnp.concatenate(
        [edge_index[1], N + pad_ids % (NPAD - N)]
    ).reshape(NW, NCHUNK, CH)
    zeros = jnp.zeros((NPAD, D), jnp.float32)
    b1r = b1.reshape(1, D)
    b2r = b2.reshape(1, D)

    agg1 = _sc_agg(x, src, dst, zeros)
    h1 = _mlp1(x, agg1, W1, b1r)
    agg2 = _sc_agg(h1, src, dst, zeros)
    return _mlp2(h1, agg2, W2, b2r)


# speedup vs baseline: 2.0052x; 1.0296x over previous
import functools

import jax
import jax.numpy as jnp
from jax import lax
from jax.experimental import pallas as pl
from jax.experimental.pallas import tpu as pltpu
from jax.experimental.pallas import tpu_sc as plsc

N = 10000
E = 320000
D = 128

NC = 2
NS = 16
NW = NC * NS
CH = 96
NCHUNK = 105
EPW = NCHUNK * CH
EPAD = NW * EPW
NPAD = 10240
RPS = NPAD // NS

_sc_mesh = plsc.VectorSubcoreMesh(core_axis_name="c", subcore_axis_name="s")


@functools.partial(
    pl.kernel,
    out_type=jax.ShapeDtypeStruct((NC, NPAD, D), jnp.float32),
    mesh=_sc_mesh,
    scratch_types=[
        pltpu.VMEM((EPW,), jnp.int32),
        pltpu.VMEM((NCHUNK, CH), jnp.int32),
        pltpu.VMEM((CH, D), jnp.float32),
        pltpu.VMEM((CH, D), jnp.float32),
        pltpu.SemaphoreType.DMA,
        pltpu.SemaphoreType.DMA,
        pltpu.VMEM_SHARED((NPAD, D), jnp.float32),
    ],
)
def _sc_agg(x_hbm, src_hbm, dst_hbm, zeros_hbm, out_hbm,
            src_v, dst_v, rows0, rows1, sem0, sem1, agg_sh):
    cid = lax.axis_index("c")
    sid = lax.axis_index("s")
    wid = cid * NS + sid
    eb = wid * EPW

    r0 = sid * RPS
    pltpu.async_copy(
        zeros_hbm.at[pl.ds(r0, RPS)], agg_sh.at[pl.ds(r0, RPS)], sem0)
    pltpu.async_copy(src_hbm.at[pl.ds(eb, EPW)], src_v, sem1)
    pltpu.sync_copy(dst_hbm.at[wid], dst_v)
    pltpu.make_async_copy(
        zeros_hbm.at[pl.ds(r0, RPS)], agg_sh.at[pl.ds(r0, RPS)], sem0).wait()
    pltpu.make_async_copy(src_hbm.at[pl.ds(eb, EPW)], src_v, sem1).wait()
    plsc.subcore_barrier()

    pltpu.async_copy(x_hbm.at[src_v.at[pl.ds(0, CH)]], rows0, sem0)
    pltpu.async_copy(x_hbm.at[src_v.at[pl.ds(CH, CH)]], rows1, sem1)

    @pl.loop(0, NCHUNK)
    def _(i):
        base = i * CH

        @pl.when(lax.rem(i, 2) == 0)
        def _():
            pltpu.make_async_copy(
                x_hbm.at[src_v.at[pl.ds(base, CH)]], rows0, sem0).wait()
            pltpu.sync_copy(rows0, agg_sh.at[dst_v.at[i]], add=True)

            @pl.when(i + 2 < NCHUNK)
            def _():
                pltpu.async_copy(
                    x_hbm.at[src_v.at[pl.ds(base + 2 * CH, CH)]], rows0, sem0)

        @pl.when(lax.rem(i, 2) == 1)
        def _():
            pltpu.make_async_copy(
                x_hbm.at[src_v.at[pl.ds(base, CH)]], rows1, sem1).wait()
            pltpu.sync_copy(rows1, agg_sh.at[dst_v.at[i]], add=True)

            @pl.when(i + 2 < NCHUNK)
            def _():
                pltpu.async_copy(
                    x_hbm.at[src_v.at[pl.ds(base + 2 * CH, CH)]], rows1, sem1)

    plsc.subcore_barrier()
    pltpu.sync_copy(agg_sh.at[pl.ds(r0, RPS)], out_hbm.at[cid, pl.ds(r0, RPS)])


_BLK = 2000


def _mlp1_body(x_ref, agg_ref, w_ref, b_ref, o_ref):
    h = x_ref[...] + agg_ref[0] + agg_ref[1]
    z = lax.dot_general(h, w_ref[...], (((1,), (1,)), ((), ())),
                        precision=lax.Precision.HIGHEST,
                        preferred_element_type=jnp.float32)
    o_ref[...] = jnp.maximum(z + b_ref[...], 0.0)


def _mlp2_body(h_ref, agg_ref, w_ref, b_ref, o_ref):
    h = h_ref[...] + agg_ref[0] + agg_ref[1]
    z = lax.dot_general(h, w_ref[...], (((1,), (1,)), ((), ())),
                        precision=lax.Precision.HIGHEST,
                        preferred_element_type=jnp.float32)
    z = z + b_ref[...]
    m = jnp.max(z, axis=1, keepdims=True)
    lse = m + jnp.log(jnp.sum(jnp.exp(z - m), axis=1, keepdims=True))
    o_ref[...] = z - lse


def _make_mlp(body):
    return pl.pallas_call(
        body,
        grid=(N // _BLK,),
        in_specs=[
            pl.BlockSpec((_BLK, D), lambda i: (i, 0)),
            pl.BlockSpec((NC, _BLK, D), lambda i: (0, i, 0)),
            pl.BlockSpec((D, D), lambda i: (0, 0)),
            pl.BlockSpec((1, D), lambda i: (0, 0)),
        ],
        out_specs=pl.BlockSpec((_BLK, D), lambda i: (i, 0)),
        out_shape=jax.ShapeDtypeStruct((N, D), jnp.float32),
    )


_mlp1 = _make_mlp(_mlp1_body)
_mlp2 = _make_mlp(_mlp2_body)


def kernel(x, edge_index, W1, b1, W2, b2):
    npad_e = EPAD - E
    pad_ids = jnp.arange(npad_e, dtype=jnp.int32)
    src = jnp.concatenate([edge_index[0], pad_ids % N])
    dst = jnp.concatenate(
        [edge_index[1], N + pad_ids % (NPAD - N)]
    ).reshape(NW, NCHUNK, CH)
    zeros = jnp.zeros((NPAD, D), jnp.float32)
    b1r = b1.reshape(1, D)
    b2r = b2.reshape(1, D)

    agg1 = _sc_agg(x, src, dst, zeros)
    h1 = _mlp1(x, agg1, W1, b1r)
    agg2 = _sc_agg(h1, src, dst, zeros)
    return _mlp2(h1, agg2, W2, b2r)
